# baseline (speedup 1.0000x reference)
import functools

import jax
import jax.numpy as jnp
from jax import lax
from jax.experimental import pallas as pl
from jax.experimental.pallas import tpu as pltpu
from jax.experimental.pallas import tpu_sc as plsc

_NC = 2
_NS = 16
_NW = _NC * _NS
_L = 16

_CHUNK = 16384
_UNROLL = 8


def _make_sc_count(n):
    per_worker = n // _NW
    n_chunks = per_worker // _CHUNK
    groups = _CHUNK // _L
    mesh = plsc.VectorSubcoreMesh(core_axis_name="c", subcore_axis_name="s")

    @functools.partial(
        pl.kernel,
        mesh=mesh,
        out_type=jax.ShapeDtypeStruct((_NW, 2, _L), jnp.int32),
        scratch_types=[
            pltpu.VMEM((2, _CHUNK), jnp.float32),
            pltpu.VMEM((2, _CHUNK), jnp.int32),
            pltpu.VMEM((2, _L), jnp.int32),
            pltpu.SemaphoreType.DMA,
            pltpu.SemaphoreType.DMA,
        ],
    )
    def sc_count(x_hbm, t_hbm, out_hbm, xbuf, tbuf, accbuf, sem0, sem1):
        wid = lax.axis_index("s") * _NC + lax.axis_index("c")
        base = wid * per_worker
        sems = (sem0, sem1)

        def copies(c, slot):
            off = base + c * _CHUNK
            return (
                pltpu.make_async_copy(
                    x_hbm.at[pl.ds(off, _CHUNK)], xbuf.at[slot], sems[slot]),
                pltpu.make_async_copy(
                    t_hbm.at[pl.ds(off, _CHUNK)], tbuf.at[slot], sems[slot]),
            )

        for cp in copies(0, 0):
            cp.start()

        zero = jnp.zeros((_L,), jnp.int32)
        acc = (zero, zero, zero, zero)
        for c in range(n_chunks):
            slot = c % 2
            if c + 1 < n_chunks:
                for cp in copies(c + 1, (c + 1) % 2):
                    cp.start()
            for cp in copies(c, slot):
                cp.wait()

            def group_body(g, gcarry, slot=slot):
                q0, q1, st0, st1 = gcarry
                for u in range(_UNROLL):
                    off = g * (_L * _UNROLL) + u * _L
                    vx = xbuf[slot, pl.ds(off, _L)]
                    vt = tbuf[slot, pl.ds(off, _L)]
                    vp = (vx + vx).astype(jnp.int32)
                    vq = vp | vt
                    if u % 2 == 0:
                        q0, st0 = q0 + vq, st0 + vt
                    else:
                        q1, st1 = q1 + vq, st1 + vt
                return q0, q1, st0, st1

            acc = lax.fori_loop(0, groups // _UNROLL, group_body, acc)

        accbuf[0, :] = acc[0] + acc[1]
        accbuf[1, :] = acc[2] + acc[3]
        pltpu.sync_copy(accbuf, out_hbm.at[wid])

    return sc_count


def kernel(inputs, targets):
    n = inputs.shape[0]
    parts = _make_sc_count(n)(inputs, targets)
    sq = parts[:, 0, :].sum()
    st = parts[:, 1, :].sum()
    tn = n - sq
    t0 = n - st
    return tn.astype(jnp.float32) / jnp.clip(t0.astype(jnp.float32), 1e-12)


# speedup vs baseline: 1.0166x; 1.0166x over previous
import functools

import jax
import jax.numpy as jnp
from jax import lax
from jax.experimental import pallas as pl
from jax.experimental.pallas import tpu as pltpu
from jax.experimental.pallas import tpu_sc as plsc

_NC = 2
_NS = 16
_NW = _NC * _NS
_L = 16

_CHUNK = 16384
_UNROLL = 16
_NACC = 4


def _make_sc_count(n):
    per_worker = n // _NW
    n_chunks = per_worker // _CHUNK
    groups = _CHUNK // _L
    mesh = plsc.VectorSubcoreMesh(core_axis_name="c", subcore_axis_name="s")

    @functools.partial(
        pl.kernel,
        mesh=mesh,
        out_type=jax.ShapeDtypeStruct((_NW, 2, _L), jnp.int32),
        scratch_types=[
            pltpu.VMEM((2, _CHUNK), jnp.float32),
            pltpu.VMEM((2, _CHUNK), jnp.int32),
            pltpu.VMEM((2, _L), jnp.int32),
            pltpu.SemaphoreType.DMA,
            pltpu.SemaphoreType.DMA,
        ],
    )
    def sc_count(x_hbm, t_hbm, out_hbm, xbuf, tbuf, accbuf, sem0, sem1):
        wid = lax.axis_index("s") * _NC + lax.axis_index("c")
        base = wid * per_worker
        sems = (sem0, sem1)

        def copies(c, slot):
            off = base + c * _CHUNK
            return (
                pltpu.make_async_copy(
                    x_hbm.at[pl.ds(off, _CHUNK)], xbuf.at[slot], sems[slot]),
                pltpu.make_async_copy(
                    t_hbm.at[pl.ds(off, _CHUNK)], tbuf.at[slot], sems[slot]),
            )

        for cp in copies(0, 0):
            cp.start()

        zero = jnp.zeros((_L,), jnp.int32)
        acc = (zero,) * (2 * _NACC)
        for c in range(n_chunks):
            slot = c % 2
            if c + 1 < n_chunks:
                for cp in copies(c + 1, (c + 1) % 2):
                    cp.start()
            for cp in copies(c, slot):
                cp.wait()

            def group_body(g, gcarry, slot=slot):
                accs = list(gcarry)
                for u in range(_UNROLL):
                    off = g * (_L * _UNROLL) + u * _L
                    vx = xbuf[slot, pl.ds(off, _L)]
                    vt = tbuf[slot, pl.ds(off, _L)]
                    m = (vx < 0.5) & (vt == 0)
                    k = u % _NACC
                    accs[k] = accs[k] + jnp.where(m, 1, 0)
                    accs[_NACC + k] = accs[_NACC + k] + vt
                return tuple(accs)

            acc = lax.fori_loop(0, groups // _UNROLL, group_body, acc)

        accbuf[0, :] = functools.reduce(lambda a, b: a + b, acc[:_NACC])
        accbuf[1, :] = functools.reduce(lambda a, b: a + b, acc[_NACC:])
        pltpu.sync_copy(accbuf, out_hbm.at[wid])

    return sc_count


def kernel(inputs, targets):
    n = inputs.shape[0]
    parts = _make_sc_count(n)(inputs, targets)
    tn = parts[:, 0, :].sum()
    st = parts[:, 1, :].sum()
    t0 = n - st
    return tn.astype(jnp.float32) / jnp.clip(t0.astype(jnp.float32), 1e-12)
